# baseline (speedup 1.0000x reference)
import functools

import jax
import jax.numpy as jnp
from jax import lax
from jax.experimental import pallas as pl
from jax.experimental.pallas import tpu as pltpu
from jax.experimental.pallas import tpu_sc as plsc

ROWS = 64
VOCAB = 100000
NUM_CORES = 2
NUM_SUBCORES = 16
NUM_GROUPS = 8
GROUP_ROWS = ROWS // NUM_GROUPS
NUM_SLABS = 4
CHUNK_W = 3584
NUM_CHUNKS = 28
CHUNKS_PER_SLAB = NUM_CHUNKS // NUM_SLABS
MAIN_COLS = 99968
LAST_OFF = MAIN_COLS - CHUNK_W
CHUNK_VREGS = CHUNK_W // 16
TAIL_W = 128
TAIL_VREGS = TAIL_W // 16

_mesh = plsc.VectorSubcoreMesh(
    core_axis_name="c", subcore_axis_name="s"
)


@functools.partial(
    pl.kernel,
    out_type=jax.ShapeDtypeStruct((NUM_GROUPS, GROUP_ROWS, 128), jnp.int32),
    mesh=_mesh,
    scratch_types=[
        pltpu.VMEM((GROUP_ROWS, CHUNK_W), jnp.float32),
        pltpu.VMEM((GROUP_ROWS, CHUNK_W), jnp.float32),
        pltpu.VMEM((GROUP_ROWS, TAIL_W), jnp.float32),
        pltpu.VMEM((GROUP_ROWS, 128), jnp.float32),
        pltpu.VMEM((GROUP_ROWS, 128), jnp.int32),
        [pltpu.VMEM((GROUP_ROWS, 128), jnp.float32)] * NUM_SLABS,
        [pltpu.VMEM((GROUP_ROWS, 128), jnp.int32)] * NUM_SLABS,
        pltpu.VMEM_SHARED((NUM_SUBCORES, GROUP_ROWS, 128), jnp.float32),
        pltpu.VMEM_SHARED((NUM_SUBCORES, GROUP_ROWS, 128), jnp.int32),
        pltpu.SemaphoreType.DMA,
        pltpu.SemaphoreType.DMA,
        pltpu.SemaphoreType.DMA,
    ],
)
def _argmax_sc(
    x_hbm,
    tail_hbm,
    out_hbm,
    buf0,
    buf1,
    tailbuf,
    resv,
    resi,
    mrgv,
    mrgi,
    shv,
    shi,
    sem0,
    sem1,
    sem2,
):
  core = lax.axis_index("c")
  tile = lax.axis_index("s")
  group = core * (NUM_GROUPS // NUM_CORES) + tile // NUM_SLABS
  slab = tile % NUM_SLABS
  row0 = group * GROUP_ROWS
  lane = lax.iota(jnp.int32, 16)

  def chunk_off(k):
    return jnp.minimum((slab + NUM_SLABS * k) * CHUNK_W, LAST_OFF)

  bufs = (buf0, buf1)
  sems = (sem0, sem1)
  copies = [None, None]
  tail_copy = pltpu.async_copy(
      tail_hbm.at[pl.ds(row0, GROUP_ROWS), :], tailbuf, sem2
  )
  copies[0] = pltpu.async_copy(
      x_hbm.at[pl.ds(row0, GROUP_ROWS), pl.ds(chunk_off(0), CHUNK_W)],
      bufs[0],
      sems[0],
  )
  accs_v = [
      jnp.full((16,), -jnp.inf, jnp.float32) for _ in range(GROUP_ROWS)
  ]
  accs_i = [jnp.zeros((16,), jnp.int32) for _ in range(GROUP_ROWS)]
  for k in range(CHUNKS_PER_SLAB):
    if k + 1 < CHUNKS_PER_SLAB:
      copies[(k + 1) % 2] = pltpu.async_copy(
          x_hbm.at[
              pl.ds(row0, GROUP_ROWS), pl.ds(chunk_off(k + 1), CHUNK_W)
          ],
          bufs[(k + 1) % 2],
          sems[(k + 1) % 2],
      )
    copies[k % 2].wait()
    buf = bufs[k % 2]
    base = chunk_off(k) + lane

    def body(i, carry, buf=buf, base=base):
      vs = list(carry[:GROUP_ROWS])
      idxs = list(carry[GROUP_ROWS:])
      idx = base + i * 16
      for r in range(GROUP_ROWS):
        v = buf[r, pl.ds(i * 16, 16)]
        m = v > vs[r]
        vs[r] = jnp.where(m, v, vs[r])
        idxs[r] = jnp.where(m, idx, idxs[r])
      return tuple(vs) + tuple(idxs)

    carry = lax.fori_loop(
        0, CHUNK_VREGS, body, tuple(accs_v) + tuple(accs_i), unroll=4
    )
    accs_v = list(carry[:GROUP_ROWS])
    accs_i = list(carry[GROUP_ROWS:])

  tail_copy.wait()
  for i in range(TAIL_VREGS):
    idx = lane + (MAIN_COLS + i * 16)
    for r in range(GROUP_ROWS):
      v = tailbuf[r, pl.ds(i * 16, 16)]
      m = v > accs_v[r]
      accs_v[r] = jnp.where(m, v, accs_v[r])
      accs_i[r] = jnp.where(m, idx, accs_i[r])

  for r in range(GROUP_ROWS):
    bv, bi = accs_v[r], accs_i[r]
    cv = bv[0]
    ci = bi[0]
    for j in range(1, 16):
      v = bv[j]
      i = bi[j]
      take = (v > cv) | ((v == cv) & (i < ci))
      cv = jnp.where(take, v, cv)
      ci = jnp.where(take, i, ci)
    resv[r, pl.ds(0, 16)] = jnp.full((16,), cv, jnp.float32)
    resi[r, pl.ds(0, 16)] = jnp.full((16,), ci, jnp.int32)
  pltpu.sync_copy(resv, shv.at[tile])
  pltpu.sync_copy(resi, shi.at[tile])
  plsc.subcore_barrier()

  @pl.when(slab == 0)
  def _merge():
    for s in range(NUM_SLABS):
      pltpu.sync_copy(shv.at[tile + s], mrgv[s])
      pltpu.sync_copy(shi.at[tile + s], mrgi[s])
    for r in range(GROUP_ROWS):
      cv = mrgv[0][r, pl.ds(0, 16)]
      ci = mrgi[0][r, pl.ds(0, 16)]
      for s in range(1, NUM_SLABS):
        v = mrgv[s][r, pl.ds(0, 16)]
        i = mrgi[s][r, pl.ds(0, 16)]
        take = (v > cv) | ((v == cv) & (i < ci))
        cv = jnp.where(take, v, cv)
        ci = jnp.where(take, i, ci)
      resi[r, pl.ds(0, 16)] = ci
    pltpu.sync_copy(resi, out_hbm.at[group])


def kernel(m_logits):
  tail = jnp.pad(
      m_logits[:, MAIN_COLS:],
      ((0, 0), (0, TAIL_W - (VOCAB - MAIN_COLS))),
      constant_values=-jnp.inf,
  )
  out = _argmax_sc(m_logits, tail)
  return out[:, :, 0].reshape(ROWS, 1)


# speedup vs baseline: 1.1800x; 1.1800x over previous
import functools

import jax
import jax.numpy as jnp
from jax import lax
from jax.experimental import pallas as pl
from jax.experimental.pallas import tpu as pltpu
from jax.experimental.pallas import tpu_sc as plsc

ROWS = 64
VOCAB = 100000
NUM_CORES = 2
NUM_SUBCORES = 16
NUM_GROUPS = 8
GROUP_ROWS = ROWS // NUM_GROUPS
NUM_SLABS = 4
CHUNK_W = 3584
NUM_CHUNKS = 28
CHUNKS_PER_SLAB = NUM_CHUNKS // NUM_SLABS
MAIN_COLS = 99968
LAST_OFF = MAIN_COLS - CHUNK_W
CHUNK_VREGS = CHUNK_W // 16
TAIL_W = 128
TAIL_VREGS = TAIL_W // 16

_mesh = plsc.VectorSubcoreMesh(
    core_axis_name="c", subcore_axis_name="s"
)


@functools.partial(
    pl.kernel,
    out_type=jax.ShapeDtypeStruct((NUM_GROUPS, GROUP_ROWS, 128), jnp.int32),
    mesh=_mesh,
    scratch_types=[
        pltpu.VMEM((GROUP_ROWS, CHUNK_W), jnp.float32),
        pltpu.VMEM((GROUP_ROWS, CHUNK_W), jnp.float32),
        pltpu.VMEM((GROUP_ROWS, TAIL_W), jnp.float32),
        pltpu.VMEM((GROUP_ROWS, 128), jnp.float32),
        pltpu.VMEM((GROUP_ROWS, 128), jnp.int32),
        [pltpu.VMEM((GROUP_ROWS, 128), jnp.float32)] * NUM_SLABS,
        [pltpu.VMEM((GROUP_ROWS, 128), jnp.int32)] * NUM_SLABS,
        pltpu.VMEM_SHARED((NUM_SUBCORES, GROUP_ROWS, 128), jnp.float32),
        pltpu.VMEM_SHARED((NUM_SUBCORES, GROUP_ROWS, 128), jnp.int32),
        pltpu.SemaphoreType.DMA,
        pltpu.SemaphoreType.DMA,
        pltpu.SemaphoreType.DMA,
    ],
)
def _argmax_sc(
    x_hbm,
    tail_hbm,
    out_hbm,
    buf0,
    buf1,
    tailbuf,
    resv,
    resi,
    mrgv,
    mrgi,
    shv,
    shi,
    sem0,
    sem1,
    sem2,
):
  core = lax.axis_index("c")
  tile = lax.axis_index("s")
  group = core * (NUM_GROUPS // NUM_CORES) + tile // NUM_SLABS
  slab = tile % NUM_SLABS
  row0 = group * GROUP_ROWS
  lane = lax.iota(jnp.int32, 16)

  def chunk_off(k):
    return jnp.minimum((slab + NUM_SLABS * k) * CHUNK_W, LAST_OFF)

  bufs = (buf0, buf1)
  sems = (sem0, sem1)
  copies = [None, None]
  tail_copy = pltpu.async_copy(
      tail_hbm.at[pl.ds(row0, GROUP_ROWS), :], tailbuf, sem2
  )
  copies[0] = pltpu.async_copy(
      x_hbm.at[pl.ds(row0, GROUP_ROWS), pl.ds(chunk_off(0), CHUNK_W)],
      bufs[0],
      sems[0],
  )
  accs_v = [
      jnp.full((16,), -jnp.inf, jnp.float32) for _ in range(GROUP_ROWS)
  ]
  accs_i = [jnp.zeros((16,), jnp.int32) for _ in range(GROUP_ROWS)]
  for k in range(CHUNKS_PER_SLAB):
    if k + 1 < CHUNKS_PER_SLAB:
      copies[(k + 1) % 2] = pltpu.async_copy(
          x_hbm.at[
              pl.ds(row0, GROUP_ROWS), pl.ds(chunk_off(k + 1), CHUNK_W)
          ],
          bufs[(k + 1) % 2],
          sems[(k + 1) % 2],
      )
    copies[k % 2].wait()
    buf = bufs[k % 2]
    base = chunk_off(k) + lane

    @plsc.parallel_loop(
        0, CHUNK_VREGS, unroll=2, carry=tuple(accs_v) + tuple(accs_i)
    )
    def carry(i, carry, buf=buf, base=base):
      vs = list(carry[:GROUP_ROWS])
      idxs = list(carry[GROUP_ROWS:])
      idx = base + i * 16
      for r in range(GROUP_ROWS):
        v = buf[r, pl.ds(i * 16, 16)]
        m = v > vs[r]
        vs[r] = jnp.where(m, v, vs[r])
        idxs[r] = jnp.where(m, idx, idxs[r])
      return tuple(vs) + tuple(idxs)
    accs_v = list(carry[:GROUP_ROWS])
    accs_i = list(carry[GROUP_ROWS:])

  tail_copy.wait()
  for i in range(TAIL_VREGS):
    idx = lane + (MAIN_COLS + i * 16)
    for r in range(GROUP_ROWS):
      v = tailbuf[r, pl.ds(i * 16, 16)]
      m = v > accs_v[r]
      accs_v[r] = jnp.where(m, v, accs_v[r])
      accs_i[r] = jnp.where(m, idx, accs_i[r])

  for r in range(GROUP_ROWS):
    bv, bi = accs_v[r], accs_i[r]
    cv = bv[0]
    ci = bi[0]
    for j in range(1, 16):
      v = bv[j]
      i = bi[j]
      take = (v > cv) | ((v == cv) & (i < ci))
      cv = jnp.where(take, v, cv)
      ci = jnp.where(take, i, ci)
    resv[r, pl.ds(0, 16)] = jnp.full((16,), cv, jnp.float32)
    resi[r, pl.ds(0, 16)] = jnp.full((16,), ci, jnp.int32)
  pltpu.sync_copy(resv, shv.at[tile])
  pltpu.sync_copy(resi, shi.at[tile])
  plsc.subcore_barrier()

  @pl.when(slab == 0)
  def _merge():
    for s in range(NUM_SLABS):
      pltpu.sync_copy(shv.at[tile + s], mrgv[s])
      pltpu.sync_copy(shi.at[tile + s], mrgi[s])
    for r in range(GROUP_ROWS):
      cv = mrgv[0][r, pl.ds(0, 16)]
      ci = mrgi[0][r, pl.ds(0, 16)]
      for s in range(1, NUM_SLABS):
        v = mrgv[s][r, pl.ds(0, 16)]
        i = mrgi[s][r, pl.ds(0, 16)]
        take = (v > cv) | ((v == cv) & (i < ci))
        cv = jnp.where(take, v, cv)
        ci = jnp.where(take, i, ci)
      resi[r, pl.ds(0, 16)] = ci
    pltpu.sync_copy(resi, out_hbm.at[group])


def kernel(m_logits):
  tail = jnp.pad(
      m_logits[:, MAIN_COLS:],
      ((0, 0), (0, TAIL_W - (VOCAB - MAIN_COLS))),
      constant_values=-jnp.inf,
  )
  out = _argmax_sc(m_logits, tail)
  return out[:, :, 0].reshape(ROWS, 1)
